# Initial kernel scaffold; baseline (speedup 1.0000x reference)
import jax
import jax.numpy as jnp
from jax.experimental import pallas as pl


def kernel(input, lengths, span_idxs, W, bias):
    raise NotImplementedError("write your pallas kernel here")


# speedup vs baseline: 21.5568x; 21.5568x over previous
import functools

import jax
import jax.numpy as jnp
from jax import lax
from jax.experimental import pallas as pl
from jax.experimental.pallas import tpu as pltpu
from jax.experimental.pallas import tpu_sc as plsc

_NUM_CORES = 2
_NUM_SUBCORES = 16
_NW = _NUM_CORES * _NUM_SUBCORES
_CHUNK = 128


def _sc_gather(table, idx3):
    nw, nch, ch = idx3.shape
    n = nch * ch
    d = table.shape[1]
    mesh = plsc.VectorSubcoreMesh(core_axis_name="c", subcore_axis_name="s")

    @functools.partial(
        pl.kernel,
        out_type=jax.ShapeDtypeStruct((nw * n, d), jnp.float32),
        mesh=mesh,
        scratch_types=[
            pltpu.VMEM((nch, ch), jnp.int32),
            pltpu.VMEM((n, d), jnp.float32),
            pltpu.SemaphoreType.DMA,
        ],
        compiler_params=pltpu.CompilerParams(use_tc_tiling_on_sc=False),
    )
    def gather_kernel(table_hbm, idx_hbm, out_hbm, idx_v, rows_v, sem):
        wid = lax.axis_index("s") * _NUM_CORES + lax.axis_index("c")
        pltpu.sync_copy(idx_hbm.at[wid], idx_v)
        copies = [
            pltpu.async_copy(
                table_hbm.at[idx_v.at[j]], rows_v.at[pl.ds(j * ch, ch)], sem
            )
            for j in range(nch)
        ]
        for c in copies:
            c.wait()
        pltpu.sync_copy(rows_v, out_hbm.at[pl.ds(wid * n, n)])

    return gather_kernel(table, idx3)


def _tc_span_matmul(E, lo, hi, bias2):
    b_dim, l_dim, d = E.shape
    s_dim = lo.shape[1]

    def body(e_ref, lo_ref, hi_ref, bias_ref, o_ref):
        pos = lax.broadcasted_iota(jnp.int32, (s_dim, l_dim), 1)
        mask = ((pos >= lo_ref[0]) & (pos < hi_ref[0])).astype(jnp.float32)
        acc = lax.dot_general(
            mask, e_ref[0], (((1,), (0,)), ((), ())),
            preferred_element_type=jnp.float32,
        )
        o_ref[0] = acc + bias_ref[...]

    return pl.pallas_call(
        body,
        grid=(b_dim,),
        in_specs=[
            pl.BlockSpec((1, l_dim, d), lambda b: (b, 0, 0)),
            pl.BlockSpec((1, s_dim, 1), lambda b: (b, 0, 0)),
            pl.BlockSpec((1, s_dim, 1), lambda b: (b, 0, 0)),
            pl.BlockSpec((1, d), lambda b: (0, 0)),
        ],
        out_specs=pl.BlockSpec((1, s_dim, d), lambda b: (b, 0, 0)),
        out_shape=jax.ShapeDtypeStruct((b_dim, s_dim, d), jnp.float32),
    )(E, lo, hi, bias2)


def kernel(input, lengths, span_idxs, W, bias):
    tokens = input
    b_dim, l_dim = tokens.shape
    s_dim = span_idxs.shape[1]
    d = W.shape[0]
    table = W.T

    t_total = b_dim * l_dim
    per_w = t_total // _NW
    idx3 = tokens.reshape(_NW, per_w // _CHUNK, _CHUNK)
    E = _sc_gather(table, idx3).reshape(b_dim, l_dim, d)

    lo = span_idxs[..., 0].reshape(b_dim, s_dim, 1)
    hi = span_idxs[..., 1].reshape(b_dim, s_dim, 1)
    return _tc_span_matmul(E, lo, hi, bias.reshape(1, d))


# speedup vs baseline: 22.7962x; 1.0575x over previous
import functools

import jax
import jax.numpy as jnp
from jax import lax
from jax.experimental import pallas as pl
from jax.experimental.pallas import tpu as pltpu
from jax.experimental.pallas import tpu_sc as plsc

_NUM_CORES = 2
_NUM_SUBCORES = 16
_NW = _NUM_CORES * _NUM_SUBCORES
_CHUNK = 128
_ROUNDS = 4


def _sc_gather(table, idx3):
    nw, nch, ch = idx3.shape
    n = nch * ch
    d = table.shape[1]
    nch_r = nch // _ROUNDS
    n_r = nch_r * ch
    mesh = plsc.VectorSubcoreMesh(core_axis_name="c", subcore_axis_name="s")

    @functools.partial(
        pl.kernel,
        out_type=jax.ShapeDtypeStruct((nw * n, d), jnp.float32),
        mesh=mesh,
        scratch_types=[
            pltpu.VMEM((nch, ch), jnp.int32),
            pltpu.VMEM((n_r, d), jnp.float32),
            pltpu.VMEM((n_r, d), jnp.float32),
            pltpu.SemaphoreType.DMA,
            pltpu.SemaphoreType.DMA,
        ],
    )
    def gather_kernel(table_hbm, idx_hbm, out_hbm, idx_v, rows_a, rows_b, sem_a, sem_b):
        wid = lax.axis_index("s") * _NUM_CORES + lax.axis_index("c")
        pltpu.sync_copy(idx_hbm.at[wid], idx_v)
        bufs = (rows_a, rows_b)
        sems = (sem_a, sem_b)
        gathers = [[] for _ in range(_ROUNDS)]
        for r in range(_ROUNDS):
            buf = bufs[r % 2]
            for j in range(nch_r):
                gathers[r].append(
                    pltpu.async_copy(
                        table_hbm.at[idx_v.at[r * nch_r + j]],
                        buf.at[pl.ds(j * ch, ch)],
                        sems[r % 2],
                    )
                )
            if r > 0:
                for c in gathers[r - 1]:
                    c.wait()
                pltpu.sync_copy(
                    bufs[(r - 1) % 2],
                    out_hbm.at[pl.ds(wid * n + (r - 1) * n_r, n_r)],
                )
        for c in gathers[_ROUNDS - 1]:
            c.wait()
        pltpu.sync_copy(
            bufs[(_ROUNDS - 1) % 2],
            out_hbm.at[pl.ds(wid * n + (_ROUNDS - 1) * n_r, n_r)],
        )

    return gather_kernel(table, idx3)


def _tc_span_matmul(E, lo, hi, tok3, bias2):
    b_dim, l_dim, d2 = E.shape
    d = d2 // 2
    s_dim = lo.shape[1]

    def body(e_ref, lo_ref, hi_ref, tok_ref, bias_ref, o_ref):
        pos = lax.broadcasted_iota(jnp.int32, (s_dim, l_dim), 1)
        span = (pos >= lo_ref[0]) & (pos < hi_ref[0])
        odd = (tok_ref[0] & 1) == 1
        mask_e = (span & jnp.logical_not(odd)).astype(jnp.float32)
        mask_o = (span & odd).astype(jnp.float32)
        e = e_ref[0]
        acc_e = lax.dot_general(
            mask_e, e, (((1,), (0,)), ((), ())),
            preferred_element_type=jnp.float32,
        )
        acc_o = lax.dot_general(
            mask_o, e, (((1,), (0,)), ((), ())),
            preferred_element_type=jnp.float32,
        )
        o_ref[0] = acc_e[:, :d] + acc_o[:, d:] + bias_ref[...]

    return pl.pallas_call(
        body,
        grid=(b_dim,),
        in_specs=[
            pl.BlockSpec((1, l_dim, d2), lambda b: (b, 0, 0)),
            pl.BlockSpec((1, s_dim, 1), lambda b: (b, 0, 0)),
            pl.BlockSpec((1, s_dim, 1), lambda b: (b, 0, 0)),
            pl.BlockSpec((1, 1, l_dim), lambda b: (b, 0, 0)),
            pl.BlockSpec((1, d), lambda b: (0, 0)),
        ],
        out_specs=pl.BlockSpec((1, s_dim, d), lambda b: (b, 0, 0)),
        out_shape=jax.ShapeDtypeStruct((b_dim, s_dim, d), jnp.float32),
    )(E, lo, hi, tok3, bias2)


def kernel(input, lengths, span_idxs, W, bias):
    tokens = input
    b_dim, l_dim = tokens.shape
    s_dim = span_idxs.shape[1]
    d = W.shape[0]
    table = W.T.reshape(W.shape[1] // 2, 2 * d)

    t_total = b_dim * l_dim
    per_w = t_total // _NW
    idx3 = (tokens >> 1).reshape(_NW, per_w // _CHUNK, _CHUNK)
    E = _sc_gather(table, idx3).reshape(b_dim, l_dim, 2 * d)

    lo = span_idxs[..., 0].reshape(b_dim, s_dim, 1)
    hi = span_idxs[..., 1].reshape(b_dim, s_dim, 1)
    tok3 = tokens.reshape(b_dim, 1, l_dim)
    return _tc_span_matmul(E, lo, hi, tok3, bias.reshape(1, d))
